# Initial kernel scaffold; baseline (speedup 1.0000x reference)
import jax
import jax.numpy as jnp
from jax.experimental import pallas as pl


def kernel(x, PD, W_in_src, a_s_src_in, a_d_src_in, W_in_snk, a_s_snk_in, a_d_snk_in, Ws_src, As_src, Ad_src, Ws_snk, As_snk, Ad_snk, src_out_W, src_out_b, snk_out_W, snk_out_b, dim1_W, dim1_b, source_edge_index, sink_edge_index):
    raise NotImplementedError("write your pallas kernel here")


# speedup vs baseline: 10.6276x; 10.6276x over previous
import functools

import jax
import jax.numpy as jnp
import numpy as np
from jax import lax
from jax.experimental import pallas as pl
from jax.experimental.pallas import tpu as pltpu
from jax.experimental.pallas import tpu_sc as plsc

N = 50000
E = 800000
HID = 32
BN = 2000
GRID = N // BN

NV = 50048
STRIPE = 3128
EP = 819200
ROWS = EP // 128
RPT = ROWS // 16
ERB = 2
BLK = RPT // ERB
DUMMY_DST = N

_f32 = jnp.float32


def _prelu(v):
    return jnp.where(v >= 0, v, 0.1 * v)


def _dot(a, b):
    return jnp.dot(a, b, preferred_element_type=_f32,
                   precision=lax.Precision.HIGHEST)


def _tca_body(comb_ref, x_ref, m_ref, w_ref, as_ref, ad_ref,
              h_ref, es_ref, ed_ref):
    m = m_ref[0, 0]
    comb = comb_ref[...]
    h = m * _dot(comb, w_ref[0]) + (1.0 - m) * (comb * x_ref[...])
    h_ref[...] = h[None]
    es_ref[...] = _dot(h, as_ref[0])[None]
    ed_ref[...] = _dot(h, ad_ref[0])[None]


def _tca(comb, x, m, w2, as2, ad2):
    return pl.pallas_call(
        _tca_body,
        grid=(2, GRID),
        in_specs=[
            pl.BlockSpec((BN, HID), lambda c, i: (i, 0)),
            pl.BlockSpec((BN, 1), lambda c, i: (i, 0)),
            pl.BlockSpec((1, 1), lambda c, i: (0, 0)),
            pl.BlockSpec((1, HID, HID), lambda c, i: (c, 0, 0)),
            pl.BlockSpec((1, HID, 1), lambda c, i: (c, 0, 0)),
            pl.BlockSpec((1, HID, 1), lambda c, i: (c, 0, 0)),
        ],
        out_specs=[
            pl.BlockSpec((1, BN, HID), lambda c, i: (c, i, 0)),
            pl.BlockSpec((1, BN, 1), lambda c, i: (c, i, 0)),
            pl.BlockSpec((1, BN, 1), lambda c, i: (c, i, 0)),
        ],
        out_shape=[
            jax.ShapeDtypeStruct((2, N, HID), _f32),
            jax.ShapeDtypeStruct((2, N, 1), _f32),
            jax.ShapeDtypeStruct((2, N, 1), _f32),
        ],
    )(comb, x, m, w2, as2, ad2)


def _tcc_body(acc_ref, den_ref, comb_ref):
    a = acc_ref[...]
    d = den_ref[...]
    e0 = jnp.concatenate([a[0, 0], a[1, 0], a[2, 0], a[3, 0]],
                         axis=1) / (d[0] + 1e-16)
    e1 = jnp.concatenate([a[0, 1], a[1, 1], a[2, 1], a[3, 1]],
                         axis=1) / (d[1] + 1e-16)
    comb_ref[...] = _prelu(e0) + _prelu(e1)


def _tcc(acc, den):
    return pl.pallas_call(
        _tcc_body,
        grid=(GRID,),
        in_specs=[
            pl.BlockSpec((4, 2, BN, 8), lambda i: (0, 0, i, 0)),
            pl.BlockSpec((2, BN, 1), lambda i: (0, i, 0)),
        ],
        out_specs=pl.BlockSpec((BN, HID), lambda i: (i, 0)),
        out_shape=jax.ShapeDtypeStruct((N, HID), _f32),
    )(acc, den)


def _tcf1_body(comb_ref, x_ref, sw_ref, sb_ref, kw_ref, kb_ref,
               fd0_ref, g_ref):
    comb = comb_ref[...]
    se = jax.nn.relu(_dot(comb, sw_ref[...]) + sb_ref[...])
    ke = jax.nn.relu(_dot(comb, kw_ref[...]) + kb_ref[...])
    fd0_ref[...] = (se + ke) * 0.5
    g_ref[...] = comb * x_ref[...]


def _tcf1(comb, x, sw, sb, kw, kb):
    return pl.pallas_call(
        _tcf1_body,
        grid=(GRID,),
        in_specs=[
            pl.BlockSpec((BN, HID), lambda i: (i, 0)),
            pl.BlockSpec((BN, 1), lambda i: (i, 0)),
            pl.BlockSpec((HID, 2), lambda i: (0, 0)),
            pl.BlockSpec((1, 2), lambda i: (0, 0)),
            pl.BlockSpec((HID, 2), lambda i: (0, 0)),
            pl.BlockSpec((1, 2), lambda i: (0, 0)),
        ],
        out_specs=[
            pl.BlockSpec((BN, 2), lambda i: (i, 0)),
            pl.BlockSpec((BN, HID), lambda i: (i, 0)),
        ],
        out_shape=[
            jax.ShapeDtypeStruct((N, 2), _f32),
            jax.ShapeDtypeStruct((N, HID), _f32),
        ],
    )(comb, x, sw, sb, kw, kb)


_PI_SCALE = np.float32(1.0 / (2.0 * np.pi * 0.01))


def _pi_partial(pts):
    b = pts[:, 0]
    p = pts[:, 1] - pts[:, 0]
    w = jnp.clip(p, 0.0, None)
    kidx = lax.broadcasted_iota(jnp.int32, (1, 25), 1)
    gx = (kidx // 5).astype(_f32) * 0.25
    gy = (kidx % 5).astype(_f32) * 0.25
    d2 = (gx - b[:, None]) ** 2 + (gy - p[:, None]) ** 2
    g = jnp.exp(-d2 / (2.0 * 0.1 ** 2)) * _PI_SCALE
    return (w[:, None] * g).sum(0)


def _tcf2_body(accc_ref, comb_ref, fd0_ref, pd_ref, dw_ref, db_ref,
               emb_ref, pia_ref, pib_ref):
    comb = comb_ref[...]
    half = comb * 0.5
    a = accc_ref[...]
    s1 = jax.nn.relu(jnp.concatenate([a[0, 0], a[1, 0], a[2, 0], a[3, 0]],
                                     axis=1) + half)
    k1 = jax.nn.relu(jnp.concatenate([a[0, 1], a[1, 1], a[2, 1], a[3, 1]],
                                     axis=1) + half)
    fd1 = jax.nn.relu(_dot(s1 + k1, dw_ref[...]) + db_ref[...])
    fd0 = fd0_ref[...]
    emb_ref[...] = jnp.concatenate([fd0, fd1], axis=1)
    pia_ref[...] = _pi_partial(fd0)[None, None]
    pib_ref[...] = _pi_partial(pd_ref[...][:, :2])[None, None]


def _tcf2(accc, comb, fd0, pd, dw, db):
    return pl.pallas_call(
        _tcf2_body,
        grid=(GRID,),
        in_specs=[
            pl.BlockSpec((4, 2, BN, 8), lambda i: (0, 0, i, 0)),
            pl.BlockSpec((BN, HID), lambda i: (i, 0)),
            pl.BlockSpec((BN, 2), lambda i: (i, 0)),
            pl.BlockSpec((BN, 4), lambda i: (i, 0)),
            pl.BlockSpec((HID, 2), lambda i: (0, 0)),
            pl.BlockSpec((1, 2), lambda i: (0, 0)),
        ],
        out_specs=[
            pl.BlockSpec((BN, 4), lambda i: (i, 0)),
            pl.BlockSpec((1, 1, 25), lambda i: (i, 0, 0)),
            pl.BlockSpec((1, 1, 25), lambda i: (i, 0, 0)),
        ],
        out_shape=[
            jax.ShapeDtypeStruct((N, 4), _f32),
            jax.ShapeDtypeStruct((GRID, 1, 25), _f32),
            jax.ShapeDtypeStruct((GRID, 1, 25), _f32),
        ],
    )(accc, comb, fd0, pd, dw, db)


@functools.lru_cache(maxsize=1)
def _build_edge_kernel():
    mesh = plsc.VectorSubcoreMesh(core_axis_name="c", subcore_axis_name="s",
                                  num_cores=2, num_subcores=16)
    return pl.kernel(
        _edge_entry,
        out_type=jax.ShapeDtypeStruct((5, 2, N, 8), _f32),
        mesh=mesh,
        compiler_params=pltpu.CompilerParams(needs_layout_passes=False,
                                             use_tc_tiling_on_sc=False),
        scratch_types=[
            pltpu.VMEM((NV,), _f32),
            pltpu.VMEM((NV,), _f32),
            pltpu.VMEM((ERB, 128), jnp.int32),
            pltpu.VMEM((ERB, 128), jnp.int32),
            pltpu.VMEM((ERB, 128), jnp.int32),
            pltpu.VMEM((ERB * 128, 8), _f32),
            pltpu.VMEM_SHARED((NV, 8), _f32),
            pltpu.SemaphoreType.DMA,
        ],
    )


def _edge_kernel(hflat, es2, ed2, edges):
    return _build_edge_kernel()(hflat, es2, ed2, edges)


def _edge_body(h_hbm, es_hbm, ed_hbm, edges_hbm, acc_out,
               es_v, ed_v, src_v, dst_v, adj_v, rows_v,
               acc_sh, gsem):
    ci = lax.axis_index("c")
    s = lax.axis_index("s")
    zero16 = jnp.zeros((16,), _f32)

    pltpu.sync_copy(es_hbm.at[pl.ds(ci * NV, NV)], es_v)
    pltpu.sync_copy(ed_hbm.at[pl.ds(ci * NV, NV)], ed_v)

    def _max_step(i, m):
        me, md = m
        return (jnp.maximum(me, es_v[pl.ds(i * 16, 16)]),
                jnp.maximum(md, ed_v[pl.ds(i * 16, 16)]))
    m_es, m_ed = lax.fori_loop(0, NV // 16, _max_step, (zero16, zero16))
    ces = m_es[0]
    ced = m_ed[0]
    for t in range(1, 16):
        ces = jnp.maximum(ces, m_es[t])
        ced = jnp.maximum(ced, m_ed[t])
    cshift = ces + ced

    lane = lax.iota(jnp.int32, 16)
    l_hi = lane >> 3
    l_lo = lane & 7
    zero16i = jnp.zeros((16,), jnp.int32)

    def _zero_buf(buf):
        def _zb(i, _):
            plsc.store_scatter(buf, [i * 2 + l_hi, l_lo], zero16)
            return 0
        lax.fori_loop(0, ERB * 64, _zb, 0)

    zoff = s * STRIPE
    LSTR = N - 15 * STRIPE
    RB = ERB * 128

    def _zero_acc_stripe():
        for k in range(STRIPE // RB):
            pltpu.sync_copy(rows_v, acc_sh.at[pl.ds(zoff + k * RB, RB)])
        pltpu.sync_copy(rows_v.at[pl.ds(0, STRIPE % RB)],
                        acc_sh.at[pl.ds(zoff + STRIPE - STRIPE % RB,
                                        STRIPE % RB)])

    _zero_buf(rows_v)
    _zero_acc_stripe()

    for p in range(5):
        plsc.subcore_barrier()
        coff = p * (2 * N) + ci * N

        def _block(b, _, p=p, coff=coff):
            chunk = s * BLK + b
            pltpu.sync_copy(edges_hbm.at[ci, 0, chunk], src_v)
            pltpu.sync_copy(edges_hbm.at[ci, 1, chunk], dst_v)

            if p < 4:
                for j in range(ERB):
                    for k in range(8):
                        adj_v[j, pl.ds(k * 16, 16)] = (
                            src_v[j, pl.ds(k * 16, 16)] + coff)
                cps = [
                    pltpu.async_copy(h_hbm.at[adj_v.at[j]],
                                     rows_v.at[pl.ds(j * 128, 128)], gsem)
                    for j in range(ERB)
                ]
                for cp in cps:
                    cp.wait()

            for j in range(ERB):
                for k in range(8):
                    sv = src_v[j, pl.ds(k * 16, 16)]
                    dv = dst_v[j, pl.ds(k * 16, 16)]
                    esg = plsc.load_gather(es_v, [sv])
                    edg = plsc.load_gather(ed_v, [dv])
                    e = esg + edg
                    e = jnp.where(e >= 0, e, 0.2 * e) - cshift
                    exv = jnp.exp(e)
                    if p == 4:
                        plsc.store_scatter(
                            rows_v, [j * 128 + k * 16 + lane, zero16i], exv)
                    else:
                        for t in range(8):
                            i = j * 128 + k * 16 + 2 * t
                            rids = i + l_hi
                            vm = jnp.where(
                                l_hi < 1,
                                jnp.full((16,), exv[2 * t], _f32),
                                jnp.full((16,), exv[2 * t + 1], _f32))
                            r = plsc.load_gather(rows_v, [rids, l_lo])
                            plsc.store_scatter(rows_v, [rids, l_lo], r * vm)

            for j in range(ERB):
                pltpu.sync_copy(rows_v.at[pl.ds(j * 128, 128)],
                                acc_sh.at[dst_v.at[j]], add=True)
            return 0

        lax.fori_loop(0, BLK, _block, 0)
        plsc.subcore_barrier()

        @pl.when(s < 15)
        def _out_full(p=p):
            pltpu.sync_copy(acc_sh.at[pl.ds(zoff, STRIPE)],
                            acc_out.at[p, ci, pl.ds(zoff, STRIPE)])

        @pl.when(s == 15)
        def _out_last(p=p):
            pltpu.sync_copy(acc_sh.at[pl.ds(15 * STRIPE, LSTR)],
                            acc_out.at[p, ci, pl.ds(15 * STRIPE, LSTR)])

        if p < 4:
            _zero_buf(rows_v)
            _zero_acc_stripe()


def _edge_entry(hflat, es2, ed2, edges, acc_out, *scratch):
    _edge_body(hflat, es2, ed2, edges, acc_out, *scratch)


def kernel(x, PD, W_in_src, a_s_src_in, a_d_src_in, W_in_snk, a_s_snk_in,
           a_d_snk_in, Ws_src, As_src, Ad_src, Ws_snk, As_snk, Ad_snk,
           src_out_W, src_out_b, snk_out_W, snk_out_b, dim1_W, dim1_b,
           source_edge_index, sink_edge_index):
    i32 = jnp.int32

    npad = EP - E
    pad = jnp.stack([jnp.zeros((npad,), i32),
                     jnp.full((npad,), DUMMY_DST, i32)])
    edges = jnp.stack([
        jnp.concatenate([source_edge_index, pad], axis=1),
        jnp.concatenate([sink_edge_index, pad], axis=1),
    ]).reshape(2, 2, ROWS // ERB, ERB, 128)

    w0 = jnp.stack([
        jnp.concatenate([W_in_src, jnp.zeros((HID - 1, HID), _f32)], axis=0),
        jnp.concatenate([W_in_snk, jnp.zeros((HID - 1, HID), _f32)], axis=0),
    ])
    as0 = jnp.stack([a_s_src_in, a_s_snk_in])[..., None]
    ad0 = jnp.stack([a_d_src_in, a_d_snk_in])[..., None]
    wl = jnp.stack([Ws_src, Ws_snk], axis=1)
    asl = jnp.stack([As_src, As_snk], axis=1)[..., None]
    adl = jnp.stack([Ad_src, Ad_snk], axis=1)[..., None]
    zw = jnp.zeros((1, 2, HID, HID), _f32)
    za = jnp.zeros((1, 2, HID, 1), _f32)
    w6 = jnp.concatenate([w0[None], wl, zw])
    as6 = jnp.concatenate([as0[None], asl, za])
    ad6 = jnp.concatenate([ad0[None], adl, za])
    m6 = jnp.array([1, 1, 1, 1, 1, 0], _f32).reshape(6, 1, 1)

    comb0 = jnp.pad(x, ((0, 0), (0, HID - 1)))

    def step(comb, inp):
        w2, as2, ad2, m = inp
        h2, es2, ed2 = _tca(comb, x, m, w2, as2, ad2)
        esf = jnp.pad(es2.reshape(2, N), ((0, 0), (0, NV - N))).reshape(-1)
        edf = jnp.pad(ed2.reshape(2, N), ((0, 0), (0, NV - N))).reshape(-1)
        hsplit = jnp.stack([h2[:, :, 0:8], h2[:, :, 8:16],
                            h2[:, :, 16:24], h2[:, :, 24:32]],
                           ).reshape(8 * N, 8)
        out5 = _edge_kernel(hsplit, esf, edf, edges)
        acc = out5[:4]
        den = out5[4, :, :, 0:1]
        comb_next = _tcc(acc, den)
        return comb_next, (acc, comb)

    _, (accs, combs) = lax.scan(step, comb0, (w6, as6, ad6, m6))

    comb_f = combs[5]
    accc = accs[5]
    fd0, _ = _tcf1(comb_f, x, src_out_W, src_out_b[None], snk_out_W,
                   snk_out_b[None])
    final_emb, pia, pib = _tcf2(accc, comb_f, fd0, PD, dim1_W, dim1_b[None])
    pi_emb = pia.sum(axis=(0, 1))
    pi_pd = pib.sum(axis=(0, 1))
    pie0 = jnp.mean((pi_pd - pi_emb) ** 2)
    return final_emb, pie0


# speedup vs baseline: 11.2094x; 1.0547x over previous
import functools

import jax
import jax.numpy as jnp
import numpy as np
from jax import lax
from jax.experimental import pallas as pl
from jax.experimental.pallas import tpu as pltpu
from jax.experimental.pallas import tpu_sc as plsc

N = 50000
E = 800000
HID = 32
BN = 2000
GRID = N // BN

NV = 50048
STRIPE = 3128
EP = 819200
ROWS = EP // 128
RPT = ROWS // 16
ERB = 2
BLK = RPT // ERB
DUMMY_DST = N

_f32 = jnp.float32


def _prelu(v):
    return jnp.where(v >= 0, v, 0.1 * v)


def _dot(a, b):
    return jnp.dot(a, b, preferred_element_type=_f32,
                   precision=lax.Precision.HIGHEST)


def _tca_body(comb_ref, x_ref, m_ref, w_ref, as_ref, ad_ref,
              h_ref, es_ref, ed_ref):
    m = m_ref[0, 0]
    comb = comb_ref[...]
    h = m * _dot(comb, w_ref[0]) + (1.0 - m) * (comb * x_ref[...])
    h_ref[...] = h[None]
    es_ref[...] = _dot(h, as_ref[0])[None]
    ed_ref[...] = _dot(h, ad_ref[0])[None]


def _tca(comb, x, m, w2, as2, ad2):
    return pl.pallas_call(
        _tca_body,
        grid=(2, GRID),
        in_specs=[
            pl.BlockSpec((BN, HID), lambda c, i: (i, 0)),
            pl.BlockSpec((BN, 1), lambda c, i: (i, 0)),
            pl.BlockSpec((1, 1), lambda c, i: (0, 0)),
            pl.BlockSpec((1, HID, HID), lambda c, i: (c, 0, 0)),
            pl.BlockSpec((1, HID, 1), lambda c, i: (c, 0, 0)),
            pl.BlockSpec((1, HID, 1), lambda c, i: (c, 0, 0)),
        ],
        out_specs=[
            pl.BlockSpec((1, BN, HID), lambda c, i: (c, i, 0)),
            pl.BlockSpec((1, BN, 1), lambda c, i: (c, i, 0)),
            pl.BlockSpec((1, BN, 1), lambda c, i: (c, i, 0)),
        ],
        out_shape=[
            jax.ShapeDtypeStruct((2, N, HID), _f32),
            jax.ShapeDtypeStruct((2, N, 1), _f32),
            jax.ShapeDtypeStruct((2, N, 1), _f32),
        ],
    )(comb, x, m, w2, as2, ad2)


def _tcc_body(acc_ref, den_ref, comb_ref):
    a = acc_ref[...]
    d = den_ref[...]
    e0 = jnp.concatenate([a[0, 0], a[1, 0], a[2, 0], a[3, 0]],
                         axis=1) / (d[0] + 1e-16)
    e1 = jnp.concatenate([a[0, 1], a[1, 1], a[2, 1], a[3, 1]],
                         axis=1) / (d[1] + 1e-16)
    comb_ref[...] = _prelu(e0) + _prelu(e1)


def _tcc(acc, den):
    return pl.pallas_call(
        _tcc_body,
        grid=(GRID,),
        in_specs=[
            pl.BlockSpec((4, 2, BN, 8), lambda i: (0, 0, i, 0)),
            pl.BlockSpec((2, BN, 1), lambda i: (0, i, 0)),
        ],
        out_specs=pl.BlockSpec((BN, HID), lambda i: (i, 0)),
        out_shape=jax.ShapeDtypeStruct((N, HID), _f32),
    )(acc, den)


def _tcf1_body(comb_ref, x_ref, sw_ref, sb_ref, kw_ref, kb_ref,
               fd0_ref, g_ref):
    comb = comb_ref[...]
    se = jax.nn.relu(_dot(comb, sw_ref[...]) + sb_ref[...])
    ke = jax.nn.relu(_dot(comb, kw_ref[...]) + kb_ref[...])
    fd0_ref[...] = (se + ke) * 0.5
    g_ref[...] = comb * x_ref[...]


def _tcf1(comb, x, sw, sb, kw, kb):
    return pl.pallas_call(
        _tcf1_body,
        grid=(GRID,),
        in_specs=[
            pl.BlockSpec((BN, HID), lambda i: (i, 0)),
            pl.BlockSpec((BN, 1), lambda i: (i, 0)),
            pl.BlockSpec((HID, 2), lambda i: (0, 0)),
            pl.BlockSpec((1, 2), lambda i: (0, 0)),
            pl.BlockSpec((HID, 2), lambda i: (0, 0)),
            pl.BlockSpec((1, 2), lambda i: (0, 0)),
        ],
        out_specs=[
            pl.BlockSpec((BN, 2), lambda i: (i, 0)),
            pl.BlockSpec((BN, HID), lambda i: (i, 0)),
        ],
        out_shape=[
            jax.ShapeDtypeStruct((N, 2), _f32),
            jax.ShapeDtypeStruct((N, HID), _f32),
        ],
    )(comb, x, sw, sb, kw, kb)


_PI_SCALE = np.float32(1.0 / (2.0 * np.pi * 0.01))


def _pi_partial(pts):
    b = pts[:, 0]
    p = pts[:, 1] - pts[:, 0]
    w = jnp.clip(p, 0.0, None)
    kidx = lax.broadcasted_iota(jnp.int32, (1, 25), 1)
    gx = (kidx // 5).astype(_f32) * 0.25
    gy = (kidx % 5).astype(_f32) * 0.25
    d2 = (gx - b[:, None]) ** 2 + (gy - p[:, None]) ** 2
    g = jnp.exp(-d2 / (2.0 * 0.1 ** 2)) * _PI_SCALE
    return (w[:, None] * g).sum(0)


def _tcf2_body(accc_ref, comb_ref, fd0_ref, pd_ref, dw_ref, db_ref,
               emb_ref, pia_ref, pib_ref):
    comb = comb_ref[...]
    half = comb * 0.5
    a = accc_ref[...]
    s1 = jax.nn.relu(jnp.concatenate([a[0, 0], a[1, 0], a[2, 0], a[3, 0]],
                                     axis=1) + half)
    k1 = jax.nn.relu(jnp.concatenate([a[0, 1], a[1, 1], a[2, 1], a[3, 1]],
                                     axis=1) + half)
    fd1 = jax.nn.relu(_dot(s1 + k1, dw_ref[...]) + db_ref[...])
    fd0 = fd0_ref[...]
    emb_ref[...] = jnp.concatenate([fd0, fd1], axis=1)
    pia_ref[...] = _pi_partial(fd0)[None, None]
    pib_ref[...] = _pi_partial(pd_ref[...][:, :2])[None, None]


def _tcf2(accc, comb, fd0, pd, dw, db):
    return pl.pallas_call(
        _tcf2_body,
        grid=(GRID,),
        in_specs=[
            pl.BlockSpec((4, 2, BN, 8), lambda i: (0, 0, i, 0)),
            pl.BlockSpec((BN, HID), lambda i: (i, 0)),
            pl.BlockSpec((BN, 2), lambda i: (i, 0)),
            pl.BlockSpec((BN, 4), lambda i: (i, 0)),
            pl.BlockSpec((HID, 2), lambda i: (0, 0)),
            pl.BlockSpec((1, 2), lambda i: (0, 0)),
        ],
        out_specs=[
            pl.BlockSpec((BN, 4), lambda i: (i, 0)),
            pl.BlockSpec((1, 1, 25), lambda i: (i, 0, 0)),
            pl.BlockSpec((1, 1, 25), lambda i: (i, 0, 0)),
        ],
        out_shape=[
            jax.ShapeDtypeStruct((N, 4), _f32),
            jax.ShapeDtypeStruct((GRID, 1, 25), _f32),
            jax.ShapeDtypeStruct((GRID, 1, 25), _f32),
        ],
    )(accc, comb, fd0, pd, dw, db)


@functools.lru_cache(maxsize=1)
def _build_edge_kernel():
    mesh = plsc.VectorSubcoreMesh(core_axis_name="c", subcore_axis_name="s",
                                  num_cores=2, num_subcores=16)
    return pl.kernel(
        _edge_entry,
        out_type=jax.ShapeDtypeStruct((5, 2, N, 8), _f32),
        mesh=mesh,
        compiler_params=pltpu.CompilerParams(needs_layout_passes=False,
                                             use_tc_tiling_on_sc=False),
        scratch_types=[
            pltpu.VMEM((NV,), _f32),
            pltpu.VMEM((NV,), _f32),
            pltpu.VMEM((ERB, 128), jnp.int32),
            pltpu.VMEM((ERB, 128), jnp.int32),
            pltpu.VMEM((ERB, 128), jnp.int32),
            pltpu.VMEM((ERB * 128, 8), _f32),
            pltpu.VMEM((ERB * 128,), _f32),
            pltpu.VMEM_SHARED((NV, 8), _f32),
            pltpu.SemaphoreType.DMA,
        ],
    )


def _edge_kernel(hflat, es2, ed2, edges):
    return _build_edge_kernel()(hflat, es2, ed2, edges)


def _edge_body(h_hbm, es_hbm, ed_hbm, edges_hbm, acc_out,
               es_v, ed_v, src_v, dst_v, adj_v, rows_v, ex_v,
               acc_sh, gsem):
    ci = lax.axis_index("c")
    s = lax.axis_index("s")
    zero16 = jnp.zeros((16,), _f32)

    pltpu.sync_copy(es_hbm.at[pl.ds(ci * NV, NV)], es_v)
    pltpu.sync_copy(ed_hbm.at[pl.ds(ci * NV, NV)], ed_v)

    def _max_step(i, m):
        me, md = m
        return (jnp.maximum(me, es_v[pl.ds(i * 16, 16)]),
                jnp.maximum(md, ed_v[pl.ds(i * 16, 16)]))
    m_es, m_ed = lax.fori_loop(0, NV // 16, _max_step, (zero16, zero16))
    ces = m_es[0]
    ced = m_ed[0]
    for t in range(1, 16):
        ces = jnp.maximum(ces, m_es[t])
        ced = jnp.maximum(ced, m_ed[t])
    cshift = ces + ced

    lane = lax.iota(jnp.int32, 16)
    l_hi = lane >> 3
    l_lo = lane & 7
    zero16i = jnp.zeros((16,), jnp.int32)

    def _zero_buf(buf):
        def _zb(i, _):
            plsc.store_scatter(buf, [i * 2 + l_hi, l_lo], zero16)
            return 0
        lax.fori_loop(0, ERB * 64, _zb, 0)

    zoff = s * STRIPE
    LSTR = N - 15 * STRIPE
    RB = ERB * 128

    def _zero_acc_stripe():
        for k in range(STRIPE // RB):
            pltpu.sync_copy(rows_v, acc_sh.at[pl.ds(zoff + k * RB, RB)])
        pltpu.sync_copy(rows_v.at[pl.ds(0, STRIPE % RB)],
                        acc_sh.at[pl.ds(zoff + STRIPE - STRIPE % RB,
                                        STRIPE % RB)])

    _zero_buf(rows_v)
    _zero_acc_stripe()

    for p in range(5):
        plsc.subcore_barrier()
        coff = p * (2 * N) + ci * N

        def _block(b, _, p=p, coff=coff):
            chunk = s * BLK + b
            pltpu.sync_copy(edges_hbm.at[ci, 0, chunk], src_v)
            pltpu.sync_copy(edges_hbm.at[ci, 1, chunk], dst_v)

            if p < 4:
                for j in range(ERB):
                    for k in range(8):
                        adj_v[j, pl.ds(k * 16, 16)] = (
                            src_v[j, pl.ds(k * 16, 16)] + coff)
                cps = [
                    pltpu.async_copy(h_hbm.at[adj_v.at[j]],
                                     rows_v.at[pl.ds(j * 128, 128)], gsem)
                    for j in range(ERB)
                ]

            for j in range(ERB):
                for k in range(8):
                    sv = src_v[j, pl.ds(k * 16, 16)]
                    dv = dst_v[j, pl.ds(k * 16, 16)]
                    esg = plsc.load_gather(es_v, [sv])
                    edg = plsc.load_gather(ed_v, [dv])
                    e = esg + edg
                    e = jnp.where(e >= 0, e, 0.2 * e) - cshift
                    exv = jnp.exp(e)
                    if p == 4:
                        plsc.store_scatter(
                            rows_v, [j * 128 + k * 16 + lane, zero16i], exv)
                    else:
                        plsc.store_scatter(ex_v, [j * 128 + k * 16 + lane],
                                           exv)

            if p < 4:
                for cp in cps:
                    cp.wait()

                def _scale(r, _):
                    ridx = 2 * r + l_hi
                    wv = plsc.load_gather(ex_v, [ridx])
                    v = plsc.load_gather(rows_v, [ridx, l_lo])
                    plsc.store_scatter(rows_v, [ridx, l_lo], v * wv)
                    return 0
                lax.fori_loop(0, RB // 2, _scale, 0)

            for j in range(ERB):
                pltpu.sync_copy(rows_v.at[pl.ds(j * 128, 128)],
                                acc_sh.at[dst_v.at[j]], add=True)
            return 0

        lax.fori_loop(0, BLK, _block, 0)
        plsc.subcore_barrier()

        @pl.when(s < 15)
        def _out_full(p=p):
            pltpu.sync_copy(acc_sh.at[pl.ds(zoff, STRIPE)],
                            acc_out.at[p, ci, pl.ds(zoff, STRIPE)])

        @pl.when(s == 15)
        def _out_last(p=p):
            pltpu.sync_copy(acc_sh.at[pl.ds(15 * STRIPE, LSTR)],
                            acc_out.at[p, ci, pl.ds(15 * STRIPE, LSTR)])

        if p < 4:
            _zero_buf(rows_v)
            _zero_acc_stripe()


def _edge_entry(hflat, es2, ed2, edges, acc_out, *scratch):
    _edge_body(hflat, es2, ed2, edges, acc_out, *scratch)


def kernel(x, PD, W_in_src, a_s_src_in, a_d_src_in, W_in_snk, a_s_snk_in,
           a_d_snk_in, Ws_src, As_src, Ad_src, Ws_snk, As_snk, Ad_snk,
           src_out_W, src_out_b, snk_out_W, snk_out_b, dim1_W, dim1_b,
           source_edge_index, sink_edge_index):
    i32 = jnp.int32

    npad = EP - E
    pad = jnp.stack([jnp.zeros((npad,), i32),
                     jnp.full((npad,), DUMMY_DST, i32)])
    edges = jnp.stack([
        jnp.concatenate([source_edge_index, pad], axis=1),
        jnp.concatenate([sink_edge_index, pad], axis=1),
    ]).reshape(2, 2, ROWS // ERB, ERB, 128)

    w0 = jnp.stack([
        jnp.concatenate([W_in_src, jnp.zeros((HID - 1, HID), _f32)], axis=0),
        jnp.concatenate([W_in_snk, jnp.zeros((HID - 1, HID), _f32)], axis=0),
    ])
    as0 = jnp.stack([a_s_src_in, a_s_snk_in])[..., None]
    ad0 = jnp.stack([a_d_src_in, a_d_snk_in])[..., None]
    wl = jnp.stack([Ws_src, Ws_snk], axis=1)
    asl = jnp.stack([As_src, As_snk], axis=1)[..., None]
    adl = jnp.stack([Ad_src, Ad_snk], axis=1)[..., None]
    zw = jnp.zeros((1, 2, HID, HID), _f32)
    za = jnp.zeros((1, 2, HID, 1), _f32)
    w6 = jnp.concatenate([w0[None], wl, zw])
    as6 = jnp.concatenate([as0[None], asl, za])
    ad6 = jnp.concatenate([ad0[None], adl, za])
    m6 = jnp.array([1, 1, 1, 1, 1, 0], _f32).reshape(6, 1, 1)

    comb0 = jnp.pad(x, ((0, 0), (0, HID - 1)))

    def step(comb, inp):
        w2, as2, ad2, m = inp
        h2, es2, ed2 = _tca(comb, x, m, w2, as2, ad2)
        esf = jnp.pad(es2.reshape(2, N), ((0, 0), (0, NV - N))).reshape(-1)
        edf = jnp.pad(ed2.reshape(2, N), ((0, 0), (0, NV - N))).reshape(-1)
        hsplit = jnp.stack([h2[:, :, 0:8], h2[:, :, 8:16],
                            h2[:, :, 16:24], h2[:, :, 24:32]],
                           ).reshape(8 * N, 8)
        out5 = _edge_kernel(hsplit, esf, edf, edges)
        acc = out5[:4]
        den = out5[4, :, :, 0:1]
        comb_next = _tcc(acc, den)
        return comb_next, (acc, comb)

    _, (accs, combs) = lax.scan(step, comb0, (w6, as6, ad6, m6))

    comb_f = combs[5]
    accc = accs[5]
    fd0, _ = _tcf1(comb_f, x, src_out_W, src_out_b[None], snk_out_W,
                   snk_out_b[None])
    final_emb, pia, pib = _tcf2(accc, comb_f, fd0, PD, dim1_W, dim1_b[None])
    pi_emb = pia.sum(axis=(0, 1))
    pi_pd = pib.sum(axis=(0, 1))
    pie0 = jnp.mean((pi_pd - pi_emb) ** 2)
    return final_emb, pie0


# speedup vs baseline: 16.7716x; 1.4962x over previous
import functools

import jax
import jax.numpy as jnp
import numpy as np
from jax import lax
from jax.experimental import pallas as pl
from jax.experimental.pallas import tpu as pltpu
from jax.experimental.pallas import tpu_sc as plsc

N = 50000
E = 800000
HID = 32
BN = 2000
GRID = N // BN

NV = 50048
STRIPE = 3128
EP = 819200
RB = 1024
CHUNKS = EP // RB
BLKN = CHUNKS // 16
DUMMY_DST = N

_f32 = jnp.float32


def _prelu(v):
    return jnp.where(v >= 0, v, 0.1 * v)


def _dot(a, b):
    return jnp.dot(a, b, preferred_element_type=_f32,
                   precision=lax.Precision.HIGHEST)


def _tca_body(comb_ref, x_ref, m_ref, w_ref, as_ref, ad_ref,
              h_ref, es_ref, ed_ref):
    m = m_ref[0, 0]
    comb = comb_ref[...]
    h = m * _dot(comb, w_ref[0]) + (1.0 - m) * (comb * x_ref[...])
    h_ref[...] = h[None]
    es_ref[...] = _dot(h, as_ref[0])[None]
    ed_ref[...] = _dot(h, ad_ref[0])[None]


def _tca(comb, x, m, w2, as2, ad2):
    return pl.pallas_call(
        _tca_body,
        grid=(2, GRID),
        in_specs=[
            pl.BlockSpec((BN, HID), lambda c, i: (i, 0)),
            pl.BlockSpec((BN, 1), lambda c, i: (i, 0)),
            pl.BlockSpec((1, 1), lambda c, i: (0, 0)),
            pl.BlockSpec((1, HID, HID), lambda c, i: (c, 0, 0)),
            pl.BlockSpec((1, HID, 1), lambda c, i: (c, 0, 0)),
            pl.BlockSpec((1, HID, 1), lambda c, i: (c, 0, 0)),
        ],
        out_specs=[
            pl.BlockSpec((1, BN, HID), lambda c, i: (c, i, 0)),
            pl.BlockSpec((1, BN, 1), lambda c, i: (c, i, 0)),
            pl.BlockSpec((1, BN, 1), lambda c, i: (c, i, 0)),
        ],
        out_shape=[
            jax.ShapeDtypeStruct((2, N, HID), _f32),
            jax.ShapeDtypeStruct((2, N, 1), _f32),
            jax.ShapeDtypeStruct((2, N, 1), _f32),
        ],
    )(comb, x, m, w2, as2, ad2)


def _tcc_body(acc_ref, den_ref, comb_ref):
    a = acc_ref[...]
    d = den_ref[...]
    e0 = jnp.concatenate([a[0, 0], a[1, 0], a[2, 0], a[3, 0]],
                         axis=1) / (d[0] + 1e-16)
    e1 = jnp.concatenate([a[0, 1], a[1, 1], a[2, 1], a[3, 1]],
                         axis=1) / (d[1] + 1e-16)
    comb_ref[...] = _prelu(e0) + _prelu(e1)


def _tcc(acc, den):
    return pl.pallas_call(
        _tcc_body,
        grid=(GRID,),
        in_specs=[
            pl.BlockSpec((4, 2, BN, 8), lambda i: (0, 0, i, 0)),
            pl.BlockSpec((2, BN, 1), lambda i: (0, i, 0)),
        ],
        out_specs=pl.BlockSpec((BN, HID), lambda i: (i, 0)),
        out_shape=jax.ShapeDtypeStruct((N, HID), _f32),
    )(acc, den)


def _tcf1_body(comb_ref, x_ref, sw_ref, sb_ref, kw_ref, kb_ref,
               fd0_ref, g_ref):
    comb = comb_ref[...]
    se = jax.nn.relu(_dot(comb, sw_ref[...]) + sb_ref[...])
    ke = jax.nn.relu(_dot(comb, kw_ref[...]) + kb_ref[...])
    fd0_ref[...] = (se + ke) * 0.5
    g_ref[...] = comb * x_ref[...]


def _tcf1(comb, x, sw, sb, kw, kb):
    return pl.pallas_call(
        _tcf1_body,
        grid=(GRID,),
        in_specs=[
            pl.BlockSpec((BN, HID), lambda i: (i, 0)),
            pl.BlockSpec((BN, 1), lambda i: (i, 0)),
            pl.BlockSpec((HID, 2), lambda i: (0, 0)),
            pl.BlockSpec((1, 2), lambda i: (0, 0)),
            pl.BlockSpec((HID, 2), lambda i: (0, 0)),
            pl.BlockSpec((1, 2), lambda i: (0, 0)),
        ],
        out_specs=[
            pl.BlockSpec((BN, 2), lambda i: (i, 0)),
            pl.BlockSpec((BN, HID), lambda i: (i, 0)),
        ],
        out_shape=[
            jax.ShapeDtypeStruct((N, 2), _f32),
            jax.ShapeDtypeStruct((N, HID), _f32),
        ],
    )(comb, x, sw, sb, kw, kb)


def _tcm_body(es_ref, ed_ref, out_ref):
    ms = jnp.maximum(jnp.max(es_ref[...]), 0.0)
    md = jnp.maximum(jnp.max(ed_ref[...]), 0.0)
    out_ref[...] = jnp.full((1, 1, 128), ms + md, _f32)


def _tcm(esf, edf):
    return pl.pallas_call(
        _tcm_body,
        grid=(2,),
        in_specs=[
            pl.BlockSpec((1, NV // 128, 128), lambda c: (c, 0, 0)),
            pl.BlockSpec((1, NV // 128, 128), lambda c: (c, 0, 0)),
        ],
        out_specs=pl.BlockSpec((1, 1, 128), lambda c: (c, 0, 0)),
        out_shape=jax.ShapeDtypeStruct((2, 1, 128), _f32),
    )(esf.reshape(2, NV // 128, 128), edf.reshape(2, NV // 128, 128))


_PI_SCALE = np.float32(1.0 / (2.0 * np.pi * 0.01))


def _pi_partial(pts):
    b = pts[:, 0]
    p = pts[:, 1] - pts[:, 0]
    w = jnp.clip(p, 0.0, None)
    kidx = lax.broadcasted_iota(jnp.int32, (1, 25), 1)
    gx = (kidx // 5).astype(_f32) * 0.25
    gy = (kidx % 5).astype(_f32) * 0.25
    d2 = (gx - b[:, None]) ** 2 + (gy - p[:, None]) ** 2
    g = jnp.exp(-d2 / (2.0 * 0.1 ** 2)) * _PI_SCALE
    return (w[:, None] * g).sum(0)


def _tcf2_body(accc_ref, comb_ref, fd0_ref, pd_ref, dw_ref, db_ref,
               emb_ref, pia_ref, pib_ref):
    comb = comb_ref[...]
    half = comb * 0.5
    a = accc_ref[...]
    s1 = jax.nn.relu(jnp.concatenate([a[0, 0], a[1, 0], a[2, 0], a[3, 0]],
                                     axis=1) + half)
    k1 = jax.nn.relu(jnp.concatenate([a[0, 1], a[1, 1], a[2, 1], a[3, 1]],
                                     axis=1) + half)
    fd1 = jax.nn.relu(_dot(s1 + k1, dw_ref[...]) + db_ref[...])
    fd0 = fd0_ref[...]
    emb_ref[...] = jnp.concatenate([fd0, fd1], axis=1)
    pia_ref[...] = _pi_partial(fd0)[None, None]
    pib_ref[...] = _pi_partial(pd_ref[...][:, :2])[None, None]


def _tcf2(accc, comb, fd0, pd, dw, db):
    return pl.pallas_call(
        _tcf2_body,
        grid=(GRID,),
        in_specs=[
            pl.BlockSpec((4, 2, BN, 8), lambda i: (0, 0, i, 0)),
            pl.BlockSpec((BN, HID), lambda i: (i, 0)),
            pl.BlockSpec((BN, 2), lambda i: (i, 0)),
            pl.BlockSpec((BN, 4), lambda i: (i, 0)),
            pl.BlockSpec((HID, 2), lambda i: (0, 0)),
            pl.BlockSpec((1, 2), lambda i: (0, 0)),
        ],
        out_specs=[
            pl.BlockSpec((BN, 4), lambda i: (i, 0)),
            pl.BlockSpec((1, 1, 25), lambda i: (i, 0, 0)),
            pl.BlockSpec((1, 1, 25), lambda i: (i, 0, 0)),
        ],
        out_shape=[
            jax.ShapeDtypeStruct((N, 4), _f32),
            jax.ShapeDtypeStruct((GRID, 1, 25), _f32),
            jax.ShapeDtypeStruct((GRID, 1, 25), _f32),
        ],
    )(accc, comb, fd0, pd, dw, db)


@functools.lru_cache(maxsize=1)
def _build_edge_kernel():
    mesh = plsc.VectorSubcoreMesh(core_axis_name="c", subcore_axis_name="s",
                                  num_cores=2, num_subcores=16)
    return pl.kernel(
        _edge_entry,
        out_type=[
            jax.ShapeDtypeStruct((5, 2, N, 8), _f32),
            jax.ShapeDtypeStruct((2, CHUNKS, RB), _f32),
        ],
        mesh=mesh,
        compiler_params=pltpu.CompilerParams(needs_layout_passes=False,
                                             use_tc_tiling_on_sc=False),
        scratch_types=[
            pltpu.VMEM((RB,), jnp.int32),
            pltpu.VMEM((RB,), jnp.int32),
            pltpu.VMEM((RB,), jnp.int32),
            pltpu.VMEM((RB,), jnp.int32),
            pltpu.VMEM((RB,), _f32),
            pltpu.VMEM((RB,), _f32),
            pltpu.VMEM((RB,), _f32),
            pltpu.VMEM((RB, 8), _f32),
            pltpu.VMEM((RB, 8), _f32),
            pltpu.VMEM((RB, 8), _f32),
            pltpu.VMEM((RB, 8), _f32),
            pltpu.VMEM((16,), _f32),
            pltpu.VMEM_SHARED((NV, 8), _f32),
            pltpu.VMEM_SHARED((NV, 8), _f32),
            pltpu.VMEM_SHARED((NV, 8), _f32),
            pltpu.SemaphoreType.DMA,
            pltpu.SemaphoreType.DMA,
            pltpu.SemaphoreType.DMA,
        ],
    )


def _edge_kernel(hflat, es2, ed2, edges, cmax):
    return _build_edge_kernel()(hflat, es2, ed2, edges, cmax)


def _edge_body(h_hbm, es_hbm, ed_hbm, edges_hbm, cmax_hbm, acc_out, ex_hbm,
               src_v, dst_v, ga_v, gb_v, esb_v, edb_v, exb_v,
               rows0_v, rows1_v, den_v, zrows_v, csh_v,
               acc0_sh, acc1_sh, dacc_sh, sem0, sem1, sem2):
    ci = lax.axis_index("c")
    s = lax.axis_index("s")
    zero16 = jnp.zeros((16,), _f32)
    lane = lax.iota(jnp.int32, 16)
    l_hi = lane >> 3
    l_lo = lane & 7
    zero16i = jnp.zeros((16,), jnp.int32)

    pltpu.sync_copy(cmax_hbm.at[ci, 0, pl.ds(0, 16)], csh_v)
    cshift = csh_v[...]

    def _zero_buf(buf):
        def _zb(i, _):
            plsc.store_scatter(buf, [i * 2 + l_hi, l_lo], zero16)
            return 0
        lax.fori_loop(0, RB // 2, _zb, 0)

    zoff = s * STRIPE

    def _zero_stripe(acc):
        for k in range(STRIPE // RB):
            pltpu.sync_copy(zrows_v, acc.at[pl.ds(zoff + k * RB, RB)])
        pltpu.sync_copy(zrows_v.at[pl.ds(0, STRIPE % RB)],
                        acc.at[pl.ds(zoff + STRIPE - STRIPE % RB,
                                     STRIPE % RB)])

    _zero_buf(zrows_v)
    _zero_buf(den_v)
    _zero_stripe(acc0_sh)
    _zero_stripe(acc1_sh)
    _zero_stripe(dacc_sh)

    def _add_off(dstref, srcref, off):
        def _ao(k, _):
            dstref[pl.ds(k * 16, 16)] = srcref[pl.ds(k * 16, 16)] + off
            return 0
        lax.fori_loop(0, RB // 16, _ao, 0)

    def _scale_rows(rows):
        def _sc(r, _):
            base = 8 * r + l_hi
            for u in range(4):
                ridx = base + 2 * u
                wv = plsc.load_gather(exb_v, [ridx])
                v = plsc.load_gather(rows, [ridx, l_lo])
                plsc.store_scatter(rows, [ridx, l_lo], v * wv)
            return 0
        lax.fori_loop(0, RB // 8, _sc, 0)

    LSTR = N - 15 * STRIPE

    def _copy_out(acc, p):
        @pl.when(s < 15)
        def _full():
            pltpu.sync_copy(acc.at[pl.ds(zoff, STRIPE)],
                            acc_out.at[p, ci, pl.ds(zoff, STRIPE)])

        @pl.when(s == 15)
        def _last():
            pltpu.sync_copy(acc.at[pl.ds(15 * STRIPE, LSTR)],
                            acc_out.at[p, ci, pl.ds(15 * STRIPE, LSTR)])

    plsc.subcore_barrier()

    def _blockA(b, _):
        chunk = s * BLKN + b
        pltpu.sync_copy(edges_hbm.at[ci, 0, chunk], src_v)
        pltpu.sync_copy(edges_hbm.at[ci, 1, chunk], dst_v)
        _add_off(ga_v, src_v, ci * NV)
        _add_off(gb_v, dst_v, ci * NV)
        cp1 = pltpu.async_copy(es_hbm.at[ga_v], esb_v, sem0)
        cp2 = pltpu.async_copy(ed_hbm.at[gb_v], edb_v, sem1)
        cp1.wait()
        cp2.wait()

        def _exc(k, _):
            e = esb_v[pl.ds(k * 16, 16)] + edb_v[pl.ds(k * 16, 16)]
            e = jnp.where(e >= 0, e, 0.2 * e) - cshift
            exb_v[pl.ds(k * 16, 16)] = jnp.exp(e)
            return 0
        lax.fori_loop(0, RB // 16, _exc, 0)
        cpe = pltpu.async_copy(exb_v, ex_hbm.at[ci, chunk], sem2)

        def _df(k, _):
            plsc.store_scatter(den_v, [k * 16 + lane, zero16i],
                               exb_v[pl.ds(k * 16, 16)])
            return 0
        lax.fori_loop(0, RB // 16, _df, 0)

        _add_off(ga_v, src_v, 0 * (2 * N) + ci * N)
        _add_off(gb_v, src_v, 1 * (2 * N) + ci * N)
        cp3 = pltpu.async_copy(h_hbm.at[ga_v], rows0_v, sem0)
        cp4 = pltpu.async_copy(h_hbm.at[gb_v], rows1_v, sem1)
        pltpu.sync_copy(den_v, dacc_sh.at[dst_v], add=True)
        cp3.wait()
        cp4.wait()
        _scale_rows(rows0_v)
        _scale_rows(rows1_v)
        pltpu.sync_copy(rows0_v, acc0_sh.at[dst_v], add=True)
        pltpu.sync_copy(rows1_v, acc1_sh.at[dst_v], add=True)
        cpe.wait()
        return 0

    lax.fori_loop(0, BLKN, _blockA, 0)
    plsc.subcore_barrier()
    _copy_out(acc0_sh, 0)
    _copy_out(acc1_sh, 1)
    _copy_out(dacc_sh, 4)
    _zero_stripe(acc0_sh)
    _zero_stripe(acc1_sh)
    plsc.subcore_barrier()

    def _blockB(b, _):
        chunk = s * BLKN + b
        pltpu.sync_copy(edges_hbm.at[ci, 0, chunk], src_v)
        pltpu.sync_copy(edges_hbm.at[ci, 1, chunk], dst_v)
        pltpu.sync_copy(ex_hbm.at[ci, chunk], exb_v)
        _add_off(ga_v, src_v, 2 * (2 * N) + ci * N)
        _add_off(gb_v, src_v, 3 * (2 * N) + ci * N)
        cp3 = pltpu.async_copy(h_hbm.at[ga_v], rows0_v, sem0)
        cp4 = pltpu.async_copy(h_hbm.at[gb_v], rows1_v, sem1)
        cp3.wait()
        cp4.wait()
        _scale_rows(rows0_v)
        _scale_rows(rows1_v)
        pltpu.sync_copy(rows0_v, acc0_sh.at[dst_v], add=True)
        pltpu.sync_copy(rows1_v, acc1_sh.at[dst_v], add=True)
        return 0

    lax.fori_loop(0, BLKN, _blockB, 0)
    plsc.subcore_barrier()
    _copy_out(acc0_sh, 2)
    _copy_out(acc1_sh, 3)


def _edge_entry(hflat, es2, ed2, edges, cmax, acc_out, ex_hbm, *scratch):
    _edge_body(hflat, es2, ed2, edges, cmax, acc_out, ex_hbm, *scratch)


def kernel(x, PD, W_in_src, a_s_src_in, a_d_src_in, W_in_snk, a_s_snk_in,
           a_d_snk_in, Ws_src, As_src, Ad_src, Ws_snk, As_snk, Ad_snk,
           src_out_W, src_out_b, snk_out_W, snk_out_b, dim1_W, dim1_b,
           source_edge_index, sink_edge_index):
    i32 = jnp.int32

    npad = EP - E
    pad = jnp.stack([jnp.zeros((npad,), i32),
                     jnp.full((npad,), DUMMY_DST, i32)])
    edges = jnp.stack([
        jnp.concatenate([source_edge_index, pad], axis=1),
        jnp.concatenate([sink_edge_index, pad], axis=1),
    ]).reshape(2, 2, CHUNKS, RB)

    w0 = jnp.stack([
        jnp.concatenate([W_in_src, jnp.zeros((HID - 1, HID), _f32)], axis=0),
        jnp.concatenate([W_in_snk, jnp.zeros((HID - 1, HID), _f32)], axis=0),
    ])
    as0 = jnp.stack([a_s_src_in, a_s_snk_in])[..., None]
    ad0 = jnp.stack([a_d_src_in, a_d_snk_in])[..., None]
    wl = jnp.stack([Ws_src, Ws_snk], axis=1)
    asl = jnp.stack([As_src, As_snk], axis=1)[..., None]
    adl = jnp.stack([Ad_src, Ad_snk], axis=1)[..., None]
    zw = jnp.zeros((1, 2, HID, HID), _f32)
    za = jnp.zeros((1, 2, HID, 1), _f32)
    w6 = jnp.concatenate([w0[None], wl, zw])
    as6 = jnp.concatenate([as0[None], asl, za])
    ad6 = jnp.concatenate([ad0[None], adl, za])
    m6 = jnp.array([1, 1, 1, 1, 1, 0], _f32).reshape(6, 1, 1)

    comb0 = jnp.pad(x, ((0, 0), (0, HID - 1)))

    def step(comb, inp):
        w2, as2, ad2, m = inp
        h2, es2, ed2 = _tca(comb, x, m, w2, as2, ad2)
        esf = jnp.pad(es2.reshape(2, N), ((0, 0), (0, NV - N))).reshape(-1)
        edf = jnp.pad(ed2.reshape(2, N), ((0, 0), (0, NV - N))).reshape(-1)
        hsplit = jnp.stack([h2[:, :, 0:8], h2[:, :, 8:16],
                            h2[:, :, 16:24], h2[:, :, 24:32]],
                           ).reshape(8 * N, 8)
        cmax = _tcm(esf, edf)
        out5, _exdump = _edge_kernel(hsplit, esf, edf, edges, cmax)
        acc = out5[:4]
        den = out5[4, :, :, 0:1]
        comb_next = _tcc(acc, den)
        return comb_next, (acc, comb)

    _, (accs, combs) = lax.scan(step, comb0, (w6, as6, ad6, m6))

    comb_f = combs[5]
    accc = accs[5]
    fd0, _ = _tcf1(comb_f, x, src_out_W, src_out_b[None], snk_out_W,
                   snk_out_b[None])
    final_emb, pia, pib = _tcf2(accc, comb_f, fd0, PD, dim1_W, dim1_b[None])
    pi_emb = pia.sum(axis=(0, 1))
    pi_pd = pib.sum(axis=(0, 1))
    pie0 = jnp.mean((pi_pd - pi_emb) ** 2)
    return final_emb, pie0
